# baseline (speedup 1.0000x reference)
import functools

import jax
import jax.numpy as jnp
from jax import lax
from jax.experimental import pallas as pl
from jax.experimental.pallas import tpu as pltpu
from jax.experimental.pallas import tpu_sc as plsc

DIM = 64
BATCH = 16384
NC = 2
NS = 16
NW = NC * NS
LANES = 16
B_PER_W = BATCH // NW
CHUNK = 128
NCHUNK = B_PER_W // CHUNK


def _transe_body(h_idx, r_idx, tp_idx, tn_idx, ent, rel, out,
                 idx_v, h_v, r_v, tp_v, tn_v, acc_v, sem):
    wid = lax.axis_index("s") * NC + lax.axis_index("c")
    base = wid * B_PER_W

    acc = jnp.zeros((LANES,), jnp.float32)
    for c in range(NCHUNK):
        off = base + c * CHUNK
        pltpu.sync_copy(h_idx.at[pl.ds(off, CHUNK)], idx_v.at[0])
        pltpu.sync_copy(r_idx.at[pl.ds(off, CHUNK)], idx_v.at[1])
        pltpu.sync_copy(tp_idx.at[pl.ds(off, CHUNK)], idx_v.at[2])
        pltpu.sync_copy(tn_idx.at[pl.ds(off, CHUNK)], idx_v.at[3])
        cp_h = pltpu.make_async_copy(ent.at[idx_v.at[0]], h_v, sem)
        cp_r = pltpu.make_async_copy(rel.at[idx_v.at[1]], r_v, sem)
        cp_tp = pltpu.make_async_copy(ent.at[idx_v.at[2]], tp_v, sem)
        cp_tn = pltpu.make_async_copy(ent.at[idx_v.at[3]], tn_v, sem)
        cp_h.start(); cp_r.start(); cp_tp.start(); cp_tn.start()
        cp_h.wait(); cp_r.wait(); cp_tp.wait(); cp_tn.wait()

        def row(j, a):
            for q in range(DIM // LANES):
                sl = pl.ds(q * LANES, LANES)
                s = h_v[j, sl] + r_v[j, sl]
                dp = s - tp_v[j, sl]
                dn = s - tn_v[j, sl]
                a = a + (dn * dn - dp * dp)
            return a

        acc = lax.fori_loop(0, CHUNK, row, acc)

    acc_v[...] = acc
    pltpu.sync_copy(acc_v, out.at[wid])


@jax.jit
def _transe_sc(h_idx, r_idx, tp_idx, tn_idx, ent, rel):
    mesh = plsc.VectorSubcoreMesh(core_axis_name="c", subcore_axis_name="s")
    grid_kernel = pl.kernel(
        _transe_body,
        out_type=jax.ShapeDtypeStruct((NW, LANES), jnp.float32),
        mesh=mesh,
        scratch_types=[
            pltpu.VMEM((4, CHUNK), jnp.int32),
            pltpu.VMEM((CHUNK, DIM), jnp.float32),
            pltpu.VMEM((CHUNK, DIM), jnp.float32),
            pltpu.VMEM((CHUNK, DIM), jnp.float32),
            pltpu.VMEM((CHUNK, DIM), jnp.float32),
            pltpu.VMEM((LANES,), jnp.float32),
            pltpu.SemaphoreType.DMA,
        ],
        compiler_params=pltpu.CompilerParams(use_tc_tiling_on_sc=False),
    )
    return grid_kernel(h_idx, r_idx, tp_idx, tn_idx, ent, rel)


def kernel(data, entity_embedding_matrix, relation_embedding_matrix):
    data = data.astype(jnp.int32)
    h_idx = data[:, 0]
    r_idx = data[:, 1]
    tp_idx = data[:, 2]
    tn_idx = data[:, 3]
    partials = _transe_sc(h_idx, r_idx, tp_idx, tn_idx,
                          entity_embedding_matrix, relation_embedding_matrix)
    return jnp.sum(partials)


# speedup vs baseline: 1.5308x; 1.5308x over previous
import jax
import jax.numpy as jnp
from jax import lax
from jax.experimental import pallas as pl
from jax.experimental.pallas import tpu as pltpu
from jax.experimental.pallas import tpu_sc as plsc

DIM = 64
BATCH = 16384
NC = 2
NS = 16
NW = NC * NS
LANES = 16
B_PER_W = BATCH // NW
G = 16
NCH = B_PER_W // G
NSTREAM = 4


def _extract(vec, q):
    return jnp.squeeze(lax.slice(vec, (q,), (q + 1,)))


def _transe_body(ent, rel, comb, out, idx_v, buf_v, acc_v, sem):
    wid = lax.axis_index("s") * NC + lax.axis_index("c")

    pltpu.sync_copy(comb.at[wid], idx_v)

    def chunk(c, acc):
        base = c * (NSTREAM * G)
        iv_h = idx_v[pl.ds(base, G)]
        iv_r = idx_v[pl.ds(base + G, G)]
        iv_tp = idx_v[pl.ds(base + 2 * G, G)]
        iv_tn = idx_v[pl.ds(base + 3 * G, G)]
        cps = []
        for s, (tab, iv) in enumerate(
                ((ent, iv_h), (rel, iv_r), (ent, iv_tp), (ent, iv_tn))):
            for q in range(G):
                cp = pltpu.make_async_copy(
                    tab.at[pl.ds(_extract(iv, q), 1)],
                    buf_v.at[pl.ds(s * G + q, 1)], sem)
                cp.start()
                cps.append(cp)
        for cp in cps:
            cp.wait()
        for q in range(G):
            for k in range(DIM // LANES):
                sl = pl.ds(k * LANES, LANES)
                s_ = buf_v[q, sl] + buf_v[G + q, sl]
                dp = s_ - buf_v[2 * G + q, sl]
                dn = s_ - buf_v[3 * G + q, sl]
                acc = acc + (dn * dn - dp * dp)
        return acc

    acc = lax.fori_loop(0, NCH, chunk, jnp.zeros((LANES,), jnp.float32))
    acc_v[...] = acc
    pltpu.sync_copy(acc_v, out.at[pl.ds(wid * LANES, LANES)])


@jax.jit
def _transe_sc(ent, rel, comb):
    mesh = plsc.VectorSubcoreMesh(core_axis_name="c", subcore_axis_name="s")
    grid_kernel = pl.kernel(
        _transe_body,
        out_type=jax.ShapeDtypeStruct((NW * LANES,), jnp.float32),
        mesh=mesh,
        scratch_types=[
            pltpu.VMEM((NCH * NSTREAM * G,), jnp.int32),
            pltpu.VMEM((NSTREAM * G, DIM), jnp.float32),
            pltpu.VMEM((LANES,), jnp.float32),
            pltpu.SemaphoreType.DMA,
        ],
    )
    return grid_kernel(ent, rel, comb)


def kernel(data, entity_embedding_matrix, relation_embedding_matrix):
    idx = data.astype(jnp.int32)
    comb = (idx.reshape(NW, NCH, G, NSTREAM)
               .transpose(0, 1, 3, 2)
               .reshape(NW, NCH * NSTREAM * G))
    partials = _transe_sc(entity_embedding_matrix, relation_embedding_matrix,
                          comb)
    return jnp.sum(partials)


# speedup vs baseline: 1.5467x; 1.0104x over previous
import jax
import jax.numpy as jnp
from jax import lax
from jax.experimental import pallas as pl
from jax.experimental.pallas import tpu as pltpu
from jax.experimental.pallas import tpu_sc as plsc

DIM = 64
BATCH = 16384
NC = 2
NS = 16
NW = NC * NS
LANES = 16
B_PER_W = BATCH // NW
G = 32
NCH = B_PER_W // G
NSTREAM = 4


def _extract(vec, q):
    return jnp.squeeze(lax.slice(vec, (q,), (q + 1,)))


def _transe_body(ent, rel, comb, out, idx_v, buf_v, acc_v, sem):
    wid = lax.axis_index("s") * NC + lax.axis_index("c")

    pltpu.sync_copy(comb.at[wid], idx_v)

    def chunk(c, acc):
        base = c * (NSTREAM * G)
        for s, tab in enumerate((ent, rel, ent, ent)):
            for sg in range(G // LANES):
                iv = idx_v[pl.ds(base + s * G + sg * LANES, LANES)]
                for q in range(LANES):
                    pltpu.make_async_copy(
                        tab.at[pl.ds(_extract(iv, q), 1)],
                        buf_v.at[pl.ds(s * G + sg * LANES + q, 1)], sem).start()
        pltpu.make_async_copy(ent.at[pl.ds(0, NSTREAM * G)], buf_v, sem).wait()
        for q in range(G):
            for k in range(DIM // LANES):
                sl = pl.ds(k * LANES, LANES)
                s_ = buf_v[q, sl] + buf_v[G + q, sl]
                dp = s_ - buf_v[2 * G + q, sl]
                dn = s_ - buf_v[3 * G + q, sl]
                acc = acc + (dn * dn - dp * dp)
        return acc

    acc = lax.fori_loop(0, NCH, chunk, jnp.zeros((LANES,), jnp.float32))
    acc_v[...] = acc
    pltpu.sync_copy(acc_v, out.at[pl.ds(wid * LANES, LANES)])


@jax.jit
def _transe_sc(ent, rel, comb):
    mesh = plsc.VectorSubcoreMesh(core_axis_name="c", subcore_axis_name="s")
    grid_kernel = pl.kernel(
        _transe_body,
        out_type=jax.ShapeDtypeStruct((NW * LANES,), jnp.float32),
        mesh=mesh,
        scratch_types=[
            pltpu.VMEM((NCH * NSTREAM * G,), jnp.int32),
            pltpu.VMEM((NSTREAM * G, DIM), jnp.float32),
            pltpu.VMEM((LANES,), jnp.float32),
            pltpu.SemaphoreType.DMA,
        ],
    )
    return grid_kernel(ent, rel, comb)


def kernel(data, entity_embedding_matrix, relation_embedding_matrix):
    idx = data.astype(jnp.int32)
    comb = (idx.reshape(NW, NCH, G, NSTREAM)
               .transpose(0, 1, 3, 2)
               .reshape(NW, NCH * NSTREAM * G))
    partials = _transe_sc(entity_embedding_matrix, relation_embedding_matrix,
                          comb)
    return jnp.sum(partials)
